# Initial kernel scaffold; baseline (speedup 1.0000x reference)
import functools

import jax
import jax.numpy as jnp
from jax import lax
from jax.experimental import pallas as pl
from jax.experimental.pallas import tpu as pltpu
from jax.experimental.pallas import tpu_sc as plsc

VOCAB = 100000
EMBED_DIM = 64
BATCH = 4096
HIST = 50

TOTAL = BATCH * HIST
NUM_CORES = 2
NUM_SUBCORES = 16
NW = NUM_CORES * NUM_SUBCORES
PER_W = TOTAL // NW
CHUNK = 800
NCHUNK = PER_W // CHUNK


def _make_kernel():
    mesh = plsc.VectorSubcoreMesh(
        core_axis_name="c", subcore_axis_name="s",
        num_cores=NUM_CORES, num_subcores=NUM_SUBCORES)

    @functools.partial(
        pl.kernel,
        out_type=jax.ShapeDtypeStruct((TOTAL, EMBED_DIM), jnp.float32),
        mesh=mesh,
        scratch_types=[
            pltpu.VMEM((NCHUNK, CHUNK), jnp.int32),
            pltpu.VMEM((CHUNK, EMBED_DIM), jnp.float32),
            pltpu.VMEM((CHUNK, EMBED_DIM), jnp.float32),
            pltpu.SemaphoreType.DMA,
            pltpu.SemaphoreType.DMA,
            pltpu.SemaphoreType.DMA,
            pltpu.SemaphoreType.DMA,
        ],
    )
    def gather_kernel(idx_hbm, table_hbm, out_hbm,
                      idx_v, rows0, rows1, gsem0, gsem1, wsem0, wsem1):
        wid = lax.axis_index("s") * NUM_CORES + lax.axis_index("c")
        base = wid * PER_W

        pltpu.sync_copy(idx_hbm.at[wid], idx_v)

        rows = (rows0, rows1)
        gsem = (gsem0, gsem1)
        wsem = (wsem0, wsem1)

        gathers = [None] * NCHUNK
        writes = [None] * NCHUNK

        gathers[0] = pltpu.async_copy(
            table_hbm.at[idx_v.at[0]], rows[0], gsem[0])
        for c in range(NCHUNK):
            b = c % 2
            gathers[c].wait()
            if c + 1 < NCHUNK:
                nb = (c + 1) % 2
                if c >= 1:
                    writes[c - 1].wait()
                gathers[c + 1] = pltpu.async_copy(
                    table_hbm.at[idx_v.at[c + 1]], rows[nb], gsem[nb])
            writes[c] = pltpu.async_copy(
                rows[b], out_hbm.at[pl.ds(base + c * CHUNK, CHUNK)], wsem[b])
        writes[NCHUNK - 2].wait()
        writes[NCHUNK - 1].wait()

    return gather_kernel


_gather = _make_kernel()


@jax.jit
def kernel(input_seq, embedding_matrix):
    idx = input_seq.reshape(NW, NCHUNK, CHUNK).astype(jnp.int32)
    out = _gather(idx, embedding_matrix)
    return out.reshape(BATCH, HIST, EMBED_DIM)


# speedup vs baseline: 4.6010x; 4.6010x over previous
import functools

import jax
import jax.numpy as jnp
from jax import lax
from jax.experimental import pallas as pl
from jax.experimental.pallas import tpu as pltpu
from jax.experimental.pallas import tpu_sc as plsc

VOCAB = 100000
EMBED_DIM = 64
BATCH = 4096
HIST = 50

TOTAL = BATCH * HIST
NUM_CORES = 2
NUM_SUBCORES = 16
NW = NUM_CORES * NUM_SUBCORES
PER_W = TOTAL // NW
CHUNK = 800
NCHUNK = PER_W // CHUNK


def _make_kernel():
    mesh = plsc.VectorSubcoreMesh(
        core_axis_name="c", subcore_axis_name="s",
        num_cores=NUM_CORES, num_subcores=NUM_SUBCORES)

    @functools.partial(
        pl.kernel,
        out_type=jax.ShapeDtypeStruct((TOTAL, EMBED_DIM), jnp.float32),
        mesh=mesh,
        compiler_params=pltpu.CompilerParams(use_tc_tiling_on_sc=False),
        scratch_types=(
            [pltpu.VMEM((CHUNK,), jnp.int32) for _ in range(NCHUNK)]
            + [
                pltpu.VMEM((CHUNK, EMBED_DIM), jnp.float32),
                pltpu.VMEM((CHUNK, EMBED_DIM), jnp.float32),
                pltpu.SemaphoreType.DMA,
                pltpu.SemaphoreType.DMA,
                pltpu.SemaphoreType.DMA,
                pltpu.SemaphoreType.DMA,
                pltpu.SemaphoreType.DMA,
            ]
        ),
    )
    def gather_kernel(idx_hbm, table_hbm, out_hbm, *scratch):
        idx_refs = scratch[:NCHUNK]
        rows0, rows1, gsem0, gsem1, wsem0, wsem1, isem = scratch[NCHUNK:]
        wid = lax.axis_index("s") * NUM_CORES + lax.axis_index("c")
        base = wid * PER_W

        idx_copies = [
            pltpu.async_copy(idx_hbm.at[wid, c], idx_refs[c], isem)
            for c in range(NCHUNK)
        ]
        for cp in idx_copies:
            cp.wait()

        rows = (rows0, rows1)
        gsem = (gsem0, gsem1)
        wsem = (wsem0, wsem1)

        gathers = [None] * NCHUNK
        writes = [None] * NCHUNK

        gathers[0] = pltpu.async_copy(
            table_hbm.at[idx_refs[0]], rows[0], gsem[0])
        for c in range(NCHUNK):
            b = c % 2
            gathers[c].wait()
            if c + 1 < NCHUNK:
                nb = (c + 1) % 2
                if c >= 1:
                    writes[c - 1].wait()
                gathers[c + 1] = pltpu.async_copy(
                    table_hbm.at[idx_refs[c + 1]], rows[nb], gsem[nb])
            writes[c] = pltpu.async_copy(
                rows[b], out_hbm.at[pl.ds(base + c * CHUNK, CHUNK)], wsem[b])
        writes[NCHUNK - 2].wait()
        writes[NCHUNK - 1].wait()

    return gather_kernel


_gather = _make_kernel()


@jax.jit
def kernel(input_seq, embedding_matrix):
    idx = input_seq.reshape(NW, NCHUNK, CHUNK).astype(jnp.int32)
    out = _gather(idx, embedding_matrix)
    return out.reshape(BATCH, HIST, EMBED_DIM)


# speedup vs baseline: 4.6683x; 1.0146x over previous
import functools

import jax
import jax.numpy as jnp
from jax import lax
from jax.experimental import pallas as pl
from jax.experimental.pallas import tpu as pltpu
from jax.experimental.pallas import tpu_sc as plsc

VOCAB = 100000
EMBED_DIM = 64
BATCH = 4096
HIST = 50

TOTAL = BATCH * HIST
NUM_CORES = 2
NUM_SUBCORES = 16
NW = NUM_CORES * NUM_SUBCORES
PER_W = TOTAL // NW
CHUNK = 400
NCHUNK = PER_W // CHUNK
NBUF = 4


def _make_kernel():
    mesh = plsc.VectorSubcoreMesh(
        core_axis_name="c", subcore_axis_name="s",
        num_cores=NUM_CORES, num_subcores=NUM_SUBCORES)

    @functools.partial(
        pl.kernel,
        out_type=jax.ShapeDtypeStruct((TOTAL, EMBED_DIM), jnp.float32),
        mesh=mesh,
        compiler_params=pltpu.CompilerParams(use_tc_tiling_on_sc=False),
        scratch_types=(
            [pltpu.VMEM((CHUNK,), jnp.int32) for _ in range(NCHUNK)]
            + [pltpu.VMEM((CHUNK, EMBED_DIM), jnp.float32)
               for _ in range(NBUF)]
            + [pltpu.SemaphoreType.DMA for _ in range(2 * NBUF + 1)]
        ),
    )
    def gather_kernel(idx_hbm, table_hbm, out_hbm, *scratch):
        idx_refs = scratch[:NCHUNK]
        rows = scratch[NCHUNK:NCHUNK + NBUF]
        gsem = scratch[NCHUNK + NBUF:NCHUNK + 2 * NBUF]
        wsem = scratch[NCHUNK + 2 * NBUF:NCHUNK + 3 * NBUF]
        isem = scratch[NCHUNK + 3 * NBUF]
        wid = lax.axis_index("s") * NUM_CORES + lax.axis_index("c")
        base = wid * PER_W

        idx_copies = [
            pltpu.async_copy(idx_hbm.at[wid, c], idx_refs[c], isem)
            for c in range(NCHUNK)
        ]
        for cp in idx_copies:
            cp.wait()

        gathers = [None] * NCHUNK
        writes = [None] * NCHUNK

        for c in range(NBUF):
            gathers[c] = pltpu.async_copy(
                table_hbm.at[idx_refs[c]], rows[c % NBUF], gsem[c % NBUF])
        for c in range(NCHUNK):
            nxt = c + NBUF - 1
            if c >= 1 and nxt < NCHUNK:
                b = (c - 1) % NBUF
                writes[c - 1].wait()
                gathers[nxt] = pltpu.async_copy(
                    table_hbm.at[idx_refs[nxt]], rows[b], gsem[b])
            gathers[c].wait()
            writes[c] = pltpu.async_copy(
                rows[c % NBUF],
                out_hbm.at[pl.ds(base + c * CHUNK, CHUNK)],
                wsem[c % NBUF])
        for c in range(max(0, NCHUNK - NBUF), NCHUNK):
            writes[c].wait()

    return gather_kernel


_gather = _make_kernel()


@jax.jit
def kernel(input_seq, embedding_matrix):
    idx = input_seq.reshape(NW, NCHUNK, CHUNK).astype(jnp.int32)
    out = _gather(idx, embedding_matrix)
    return out.reshape(BATCH, HIST, EMBED_DIM)
